# baseline (speedup 1.0000x reference)
import functools

import jax
import jax.numpy as jnp
from jax import lax
from jax.experimental import pallas as pl
from jax.experimental.pallas import tpu as pltpu
from jax.experimental.pallas import tpu_sc as plsc

N = 320000
NUM_SEG = 10000
D = 128
ZMAX = 100

NUM_TILES = 16
LANES = 128
ROWS = 2528
N_PAD = ROWS * LANES
R = 32
NB = ROWS // R
K_WIN = ROWS // NUM_TILES
N_ROWS_REAL = N // LANES


B = R * LANES


def _atom_scalar_body(z_ref, p3_ref, emb_ref, wp_ref,
                      wg_ref, w2_ref, y_ref):
    embT_h = (emb_ref[...].T * 0.5).astype(jnp.bfloat16)
    wpT_h = wp_ref[...].T * 0.5
    vT = (wg_ref[...] * w2_ref[...]).T
    types = lax.broadcasted_iota(jnp.int32, (ZMAX, LANES), 0)

    oh = jnp.concatenate(
        [(types == z_ref[g:g + 1, :]) for g in range(R)],
        axis=1).astype(jnp.bfloat16)
    pose = jnp.concatenate(
        [jnp.concatenate([p3_ref[k, g:g + 1, :] for g in range(R)], axis=1)
         for k in range(3)], axis=0)

    h = lax.dot_general(wpT_h, pose, (((1,), (0,)), ((), ())),
                        preferred_element_type=jnp.float32
                        ).astype(jnp.bfloat16)
    t = lax.tanh(h)
    lift_h = h * (t * jnp.bfloat16(0.5) + jnp.bfloat16(0.5))
    eg = lax.dot_general(embT_h, oh, (((1,), (0,)), ((), ())),
                         preferred_element_type=jnp.float32
                         ).astype(jnp.bfloat16)
    h2 = eg + lift_h
    su = h2 * lax.tanh(h2) + h2
    y = lax.dot_general(vT.astype(jnp.bfloat16), su,
                        (((1,), (0,)), ((), ())),
                        preferred_element_type=jnp.float32)
    y32 = jnp.concatenate(
        [y[:, g * LANES:(g + 1) * LANES] for g in range(R)], axis=0)
    grow = pl.program_id(0) * R + lax.broadcasted_iota(jnp.int32, (R, 1), 0)
    y_ref[...] = jnp.where(grow < N_ROWS_REAL, y32, 0.0)


def _atom_scalars(z2, p3, emb, wp, wg, w2):
    return pl.pallas_call(
        _atom_scalar_body,
        grid=(NB,),
        in_specs=[
            pl.BlockSpec((R, LANES), lambda i: (i, 0)),
            pl.BlockSpec((3, R, LANES), lambda i: (0, i, 0)),
            pl.BlockSpec((ZMAX, D), lambda i: (0, 0)),
            pl.BlockSpec((3, D), lambda i: (0, 0)),
            pl.BlockSpec((D, 1), lambda i: (0, 0)),
            pl.BlockSpec((D, 1), lambda i: (0, 0)),
        ],
        out_specs=pl.BlockSpec((R, LANES), lambda i: (i, 0)),
        out_shape=jax.ShapeDtypeStruct((ROWS, LANES), jnp.float32),
    )(z2, p3, emb, wp, wg, w2)


def _segsum_body(y_hbm, idx_hbm, init_hbm, out_hbm, yv, iv, acc):
    s = lax.axis_index("s")

    pltpu.sync_copy(y_hbm.at[s], yv)
    pltpu.sync_copy(idx_hbm.at[s], iv)

    @pl.when(s == 0)
    def _():
        pltpu.sync_copy(init_hbm, acc)

    plsc.subcore_barrier()

    pltpu.sync_copy(yv, acc.at[iv], add=True)

    plsc.subcore_barrier()

    @pl.when(s == 0)
    def _():
        pltpu.sync_copy(acc, out_hbm)


@functools.cache
def _build_segsum():
    return pl.kernel(
        _segsum_body,
        out_type=jax.ShapeDtypeStruct((NUM_SEG,), jnp.float32),
        mesh=plsc.VectorSubcoreMesh(core_axis_name="c", subcore_axis_name="s",
                                    num_cores=1, num_subcores=NUM_TILES),
        scratch_types=[
            pltpu.VMEM((K_WIN * LANES,), jnp.float32),
            pltpu.VMEM((K_WIN * LANES,), jnp.int32),
            pltpu.VMEM_SHARED((NUM_SEG,), jnp.float32),
        ],
    )


def kernel(z, pos, batch, embedding, Wp, w_gate, W2, b2):
    pad = N_PAD - N
    z2 = jnp.pad(z.astype(jnp.int32), (0, pad)).reshape(ROWS, LANES)
    p3 = jnp.pad(pos.T, ((0, 0), (0, pad))).reshape(3, ROWS, LANES)
    wg = w_gate.reshape(D, 1)

    y = _atom_scalars(z2, p3, embedding, Wp, wg, W2)

    y3 = y.reshape(NUM_TILES, LANES, K_WIN).swapaxes(1, 2).reshape(
        NUM_TILES, K_WIN * LANES)
    idx3 = jnp.pad(batch.astype(jnp.int32), (0, pad)).reshape(
        NUM_TILES, LANES, K_WIN).swapaxes(1, 2).reshape(
        NUM_TILES, K_WIN * LANES)
    init = jnp.broadcast_to(b2, (NUM_SEG,)).astype(jnp.float32)

    out = _build_segsum()(y3, idx3, init)
    return out.reshape(NUM_SEG, 1)


# speedup vs baseline: 1.0049x; 1.0049x over previous
import functools

import jax
import jax.numpy as jnp
from jax import lax
from jax.experimental import pallas as pl
from jax.experimental.pallas import tpu as pltpu
from jax.experimental.pallas import tpu_sc as plsc

N = 320000
NUM_SEG = 10000
D = 128
ZMAX = 100

NUM_TILES = 16
LANES = 128
ROWS = 2560
N_PAD = ROWS * LANES
R = 32
HALF_ROWS = ROWS // 2
NB_H = HALF_ROWS // R
ELEMS = HALF_ROWS * LANES // NUM_TILES
N_ROWS_REAL = N // LANES
B = R * LANES


def _atom_scalar_body(row0_ref, z_ref, p3_ref, emb_ref, wp_ref,
                      wg_ref, w2_ref, y_ref):
    embT_h = (emb_ref[...].T * 0.5).astype(jnp.bfloat16)
    wpT_h = wp_ref[...].T * 0.5
    vT = (wg_ref[...] * w2_ref[...]).T
    types = lax.broadcasted_iota(jnp.int32, (ZMAX, LANES), 0)

    oh = jnp.concatenate(
        [(types == z_ref[g:g + 1, :]) for g in range(R)],
        axis=1).astype(jnp.bfloat16)
    pose = jnp.concatenate(
        [jnp.concatenate([p3_ref[k, g:g + 1, :] for g in range(R)], axis=1)
         for k in range(3)], axis=0)

    h = lax.dot_general(wpT_h, pose, (((1,), (0,)), ((), ())),
                        preferred_element_type=jnp.float32
                        ).astype(jnp.bfloat16)
    t = lax.tanh(h)
    lift_h = h * (t * jnp.bfloat16(0.5) + jnp.bfloat16(0.5))
    eg = lax.dot_general(embT_h, oh, (((1,), (0,)), ((), ())),
                         preferred_element_type=jnp.float32
                         ).astype(jnp.bfloat16)
    h2 = eg + lift_h
    su = h2 * lax.tanh(h2) + h2
    y = lax.dot_general(vT.astype(jnp.bfloat16), su,
                        (((1,), (0,)), ((), ())),
                        preferred_element_type=jnp.float32)
    y32 = jnp.concatenate(
        [y[:, g * LANES:(g + 1) * LANES] for g in range(R)], axis=0)
    grow = (row0_ref[0] + pl.program_id(0) * R
            + lax.broadcasted_iota(jnp.int32, (R, 1), 0))
    y_ref[...] = jnp.where(grow < N_ROWS_REAL, y32, 0.0)


def _atom_scalars(row0, z2, p3, emb, wp, wg, w2):
    return pl.pallas_call(
        _atom_scalar_body,
        grid=(NB_H,),
        in_specs=[
            pl.BlockSpec(memory_space=pltpu.SMEM),
            pl.BlockSpec((R, LANES), lambda i: (i, 0)),
            pl.BlockSpec((3, R, LANES), lambda i: (0, i, 0)),
            pl.BlockSpec((ZMAX, D), lambda i: (0, 0)),
            pl.BlockSpec((3, D), lambda i: (0, 0)),
            pl.BlockSpec((D, 1), lambda i: (0, 0)),
            pl.BlockSpec((D, 1), lambda i: (0, 0)),
        ],
        out_specs=pl.BlockSpec((R, LANES), lambda i: (i, 0)),
        out_shape=jax.ShapeDtypeStruct((HALF_ROWS, LANES), jnp.float32),
    )(row0, z2, p3, emb, wp, wg, w2)


def _segsum_body(y_hbm, idx_hbm, init_hbm, out_hbm, yv, iv, acc):
    s = lax.axis_index("s")

    pltpu.sync_copy(y_hbm.at[s], yv)
    pltpu.sync_copy(idx_hbm.at[s], iv)

    @pl.when(s == 0)
    def _():
        pltpu.sync_copy(init_hbm, acc)

    plsc.subcore_barrier()

    pltpu.sync_copy(yv, acc.at[iv], add=True)

    plsc.subcore_barrier()

    @pl.when(s == 0)
    def _():
        pltpu.sync_copy(acc, out_hbm)


@functools.cache
def _build_segsum():
    return pl.kernel(
        _segsum_body,
        out_type=jax.ShapeDtypeStruct((NUM_SEG,), jnp.float32),
        mesh=plsc.VectorSubcoreMesh(core_axis_name="c", subcore_axis_name="s",
                                    num_cores=1, num_subcores=NUM_TILES),
        scratch_types=[
            pltpu.VMEM((ELEMS,), jnp.float32),
            pltpu.VMEM((ELEMS,), jnp.int32),
            pltpu.VMEM_SHARED((NUM_SEG,), jnp.float32),
        ],
    )


def kernel(z, pos, batch, embedding, Wp, w_gate, W2, b2):
    pad = N_PAD - N
    z2 = jnp.pad(z.astype(jnp.int32), (0, pad)).reshape(ROWS, LANES)
    p3 = jnp.pad(pos.T, ((0, 0), (0, pad))).reshape(3, ROWS, LANES)
    wg = w_gate.reshape(D, 1)
    idx2 = jnp.pad(batch.astype(jnp.int32), (0, pad)).reshape(
        2, NUM_TILES, ELEMS)
    init = jnp.broadcast_to(b2, (NUM_SEG,)).astype(jnp.float32)
    segsum = _build_segsum()

    row0_a = jnp.zeros((1,), jnp.int32)
    row0_b = jnp.full((1,), HALF_ROWS, jnp.int32)

    y_a = _atom_scalars(row0_a, z2[:HALF_ROWS], p3[:, :HALF_ROWS],
                        embedding, Wp, wg, W2)
    part = segsum(y_a.reshape(NUM_TILES, ELEMS), idx2[0], init)
    y_b = _atom_scalars(row0_b, z2[HALF_ROWS:], p3[:, HALF_ROWS:],
                        embedding, Wp, wg, W2)
    out = segsum(y_b.reshape(NUM_TILES, ELEMS), idx2[1], part)
    return out.reshape(NUM_SEG, 1)


# speedup vs baseline: 1.0464x; 1.0413x over previous
import functools

import jax
import jax.numpy as jnp
from jax import lax
from jax.experimental import pallas as pl
from jax.experimental.pallas import tpu as pltpu
from jax.experimental.pallas import tpu_sc as plsc

N = 320000
NUM_SEG = 10000
D = 128
ZMAX = 100

NUM_TILES = 16
LANES = 128
ROWS = 2560
N_PAD = ROWS * LANES
R = 64
NB = ROWS // R
K_WIN = ROWS // NUM_TILES
N_ROWS_REAL = N // LANES


B = R * LANES


def _atom_scalar_body(z_ref, p3_ref, emb_ref, wp_ref,
                      wg_ref, w2_ref, y_ref):
    embT_h = (emb_ref[...].T * 0.5).astype(jnp.bfloat16)
    wpT_h = wp_ref[...].T * 0.5
    vT = (wg_ref[...] * w2_ref[...]).T
    types = lax.broadcasted_iota(jnp.int32, (ZMAX, LANES), 0)

    oh = jnp.concatenate(
        [(types == z_ref[g:g + 1, :]) for g in range(R)],
        axis=1).astype(jnp.bfloat16)
    pose = jnp.concatenate(
        [jnp.concatenate([p3_ref[k, g:g + 1, :] for g in range(R)], axis=1)
         for k in range(3)], axis=0)

    h = lax.dot_general(wpT_h, pose, (((1,), (0,)), ((), ())),
                        preferred_element_type=jnp.float32
                        ).astype(jnp.bfloat16)
    t = lax.tanh(h)
    lift_h = h * (t * jnp.bfloat16(0.5) + jnp.bfloat16(0.5))
    eg = lax.dot_general(embT_h, oh, (((1,), (0,)), ((), ())),
                         preferred_element_type=jnp.float32
                         ).astype(jnp.bfloat16)
    h2 = eg + lift_h
    su = h2 * lax.tanh(h2) + h2
    y = lax.dot_general(vT.astype(jnp.bfloat16), su,
                        (((1,), (0,)), ((), ())),
                        preferred_element_type=jnp.float32)
    y32 = jnp.concatenate(
        [y[:, g * LANES:(g + 1) * LANES] for g in range(R)], axis=0)
    grow = pl.program_id(0) * R + lax.broadcasted_iota(jnp.int32, (R, 1), 0)
    y_ref[...] = jnp.where(grow < N_ROWS_REAL, y32, 0.0)


def _atom_scalars(z2, p3, emb, wp, wg, w2):
    return pl.pallas_call(
        _atom_scalar_body,
        grid=(NB,),
        in_specs=[
            pl.BlockSpec((R, LANES), lambda i: (i, 0)),
            pl.BlockSpec((3, R, LANES), lambda i: (0, i, 0)),
            pl.BlockSpec((ZMAX, D), lambda i: (0, 0)),
            pl.BlockSpec((3, D), lambda i: (0, 0)),
            pl.BlockSpec((D, 1), lambda i: (0, 0)),
            pl.BlockSpec((D, 1), lambda i: (0, 0)),
        ],
        out_specs=pl.BlockSpec((R, LANES), lambda i: (i, 0)),
        out_shape=jax.ShapeDtypeStruct((ROWS, LANES), jnp.float32),
    )(z2, p3, emb, wp, wg, w2)


def _segsum_body(y_hbm, idx_hbm, init_hbm, out_hbm, yv, iv, acc):
    s = lax.axis_index("s")

    pltpu.sync_copy(y_hbm.at[s], yv)
    pltpu.sync_copy(idx_hbm.at[s], iv)

    @pl.when(s == 0)
    def _():
        pltpu.sync_copy(init_hbm, acc)

    plsc.subcore_barrier()

    pltpu.sync_copy(yv, acc.at[iv], add=True)

    plsc.subcore_barrier()

    @pl.when(s == 0)
    def _():
        pltpu.sync_copy(acc, out_hbm)


@functools.cache
def _build_segsum():
    return pl.kernel(
        _segsum_body,
        out_type=jax.ShapeDtypeStruct((NUM_SEG,), jnp.float32),
        mesh=plsc.VectorSubcoreMesh(core_axis_name="c", subcore_axis_name="s",
                                    num_cores=1, num_subcores=NUM_TILES),
        scratch_types=[
            pltpu.VMEM((K_WIN * LANES,), jnp.float32),
            pltpu.VMEM((K_WIN * LANES,), jnp.int32),
            pltpu.VMEM_SHARED((NUM_SEG,), jnp.float32),
        ],
    )


def kernel(z, pos, batch, embedding, Wp, w_gate, W2, b2):
    pad = N_PAD - N
    z2 = jnp.pad(z.astype(jnp.int32), (0, pad)).reshape(ROWS, LANES)
    p3 = jnp.pad(pos.T, ((0, 0), (0, pad))).reshape(3, ROWS, LANES)
    wg = w_gate.reshape(D, 1)

    y = _atom_scalars(z2, p3, embedding, Wp, wg, W2)

    y3 = y.reshape(NUM_TILES, K_WIN * LANES)
    idx3 = jnp.pad(batch.astype(jnp.int32), (0, pad)).reshape(
        NUM_TILES, K_WIN * LANES)
    init = jnp.broadcast_to(b2, (NUM_SEG,)).astype(jnp.float32)

    out = _build_segsum()(y3, idx3, init)
    return out.reshape(NUM_SEG, 1)


# speedup vs baseline: 1.0822x; 1.0342x over previous
import functools

import jax
import jax.numpy as jnp
from jax import lax
from jax.experimental import pallas as pl
from jax.experimental.pallas import tpu as pltpu
from jax.experimental.pallas import tpu_sc as plsc

N = 320000
NUM_SEG = 10000
D = 128
ZMAX = 100

NUM_TILES = 16
LANES = 128
ROWS = 2560
N_PAD = ROWS * LANES
R = 128
NB = ROWS // R
K_WIN = ROWS // NUM_TILES
N_ROWS_REAL = N // LANES


B = R * LANES


def _atom_scalar_body(z_ref, p3_ref, emb_ref, wp_ref,
                      wg_ref, w2_ref, y_ref):
    embT_h = (emb_ref[...].T * 0.5).astype(jnp.bfloat16)
    wpT_h = wp_ref[...].T * 0.5
    vT = (wg_ref[...] * w2_ref[...]).T
    types = lax.broadcasted_iota(jnp.int32, (ZMAX, LANES), 0)

    oh = jnp.concatenate(
        [(types == z_ref[g:g + 1, :]) for g in range(R)],
        axis=1).astype(jnp.bfloat16)
    pose = jnp.concatenate(
        [jnp.concatenate([p3_ref[k, g:g + 1, :] for g in range(R)], axis=1)
         for k in range(3)], axis=0)

    h = lax.dot_general(wpT_h, pose, (((1,), (0,)), ((), ())),
                        preferred_element_type=jnp.float32
                        ).astype(jnp.bfloat16)
    t = lax.tanh(h)
    lift_h = h * (t * jnp.bfloat16(0.5) + jnp.bfloat16(0.5))
    eg = lax.dot_general(embT_h, oh, (((1,), (0,)), ((), ())),
                         preferred_element_type=jnp.float32
                         ).astype(jnp.bfloat16)
    h2 = eg + lift_h
    su = h2 * lax.tanh(h2) + h2
    y = lax.dot_general(vT.astype(jnp.bfloat16), su,
                        (((1,), (0,)), ((), ())),
                        preferred_element_type=jnp.float32)
    y32 = jnp.concatenate(
        [y[:, g * LANES:(g + 1) * LANES] for g in range(R)], axis=0)
    grow = pl.program_id(0) * R + lax.broadcasted_iota(jnp.int32, (R, 1), 0)
    y_ref[...] = jnp.where(grow < N_ROWS_REAL, y32, 0.0)


def _atom_scalars(z2, p3, emb, wp, wg, w2):
    return pl.pallas_call(
        _atom_scalar_body,
        grid=(NB,),
        in_specs=[
            pl.BlockSpec((R, LANES), lambda i: (i, 0)),
            pl.BlockSpec((3, R, LANES), lambda i: (0, i, 0)),
            pl.BlockSpec((ZMAX, D), lambda i: (0, 0)),
            pl.BlockSpec((3, D), lambda i: (0, 0)),
            pl.BlockSpec((D, 1), lambda i: (0, 0)),
            pl.BlockSpec((D, 1), lambda i: (0, 0)),
        ],
        out_specs=pl.BlockSpec((R, LANES), lambda i: (i, 0)),
        out_shape=jax.ShapeDtypeStruct((ROWS, LANES), jnp.float32),
    )(z2, p3, emb, wp, wg, w2)


def _segsum_body(y_hbm, idx_hbm, init_hbm, out_hbm, yv, iv, acc):
    s = lax.axis_index("s")

    pltpu.sync_copy(y_hbm.at[s], yv)
    pltpu.sync_copy(idx_hbm.at[s], iv)

    @pl.when(s == 0)
    def _():
        pltpu.sync_copy(init_hbm, acc)

    plsc.subcore_barrier()

    pltpu.sync_copy(yv, acc.at[iv], add=True)

    plsc.subcore_barrier()

    @pl.when(s == 0)
    def _():
        pltpu.sync_copy(acc, out_hbm)


@functools.cache
def _build_segsum():
    return pl.kernel(
        _segsum_body,
        out_type=jax.ShapeDtypeStruct((NUM_SEG,), jnp.float32),
        mesh=plsc.VectorSubcoreMesh(core_axis_name="c", subcore_axis_name="s",
                                    num_cores=1, num_subcores=NUM_TILES),
        scratch_types=[
            pltpu.VMEM((K_WIN * LANES,), jnp.float32),
            pltpu.VMEM((K_WIN * LANES,), jnp.int32),
            pltpu.VMEM_SHARED((NUM_SEG,), jnp.float32),
        ],
    )


def kernel(z, pos, batch, embedding, Wp, w_gate, W2, b2):
    pad = N_PAD - N
    z2 = jnp.pad(z.astype(jnp.int32), (0, pad)).reshape(ROWS, LANES)
    p3 = jnp.pad(pos.T, ((0, 0), (0, pad))).reshape(3, ROWS, LANES)
    wg = w_gate.reshape(D, 1)

    y = _atom_scalars(z2, p3, embedding, Wp, wg, W2)

    y3 = y.reshape(NUM_TILES, K_WIN * LANES)
    idx3 = jnp.pad(batch.astype(jnp.int32), (0, pad)).reshape(
        NUM_TILES, K_WIN * LANES)
    init = jnp.broadcast_to(b2, (NUM_SEG,)).astype(jnp.float32)

    out = _build_segsum()(y3, idx3, init)
    return out.reshape(NUM_SEG, 1)


# speedup vs baseline: 1.1003x; 1.0166x over previous
import functools

import jax
import jax.numpy as jnp
from jax import lax
from jax.experimental import pallas as pl
from jax.experimental.pallas import tpu as pltpu
from jax.experimental.pallas import tpu_sc as plsc

N = 320000
NUM_SEG = 10000
D = 128
ZMAX = 100

NUM_TILES = 16
LANES = 128
ROWS = 2560
N_PAD = ROWS * LANES
R = 256
NB = ROWS // R
K_WIN = ROWS // NUM_TILES
N_ROWS_REAL = N // LANES


B = R * LANES


def _atom_scalar_body(z_ref, p3_ref, emb_ref, wp_ref,
                      wg_ref, w2_ref, y_ref):
    embT_h = (emb_ref[...].T * 0.5).astype(jnp.bfloat16)
    wpT_h = wp_ref[...].T * 0.5
    vT = (wg_ref[...] * w2_ref[...]).T
    types = lax.broadcasted_iota(jnp.int32, (ZMAX, LANES), 0)

    oh = jnp.concatenate(
        [(types == z_ref[g:g + 1, :]) for g in range(R)],
        axis=1).astype(jnp.bfloat16)
    pose = jnp.concatenate(
        [jnp.concatenate([p3_ref[k, g:g + 1, :] for g in range(R)], axis=1)
         for k in range(3)], axis=0)

    h = lax.dot_general(wpT_h, pose, (((1,), (0,)), ((), ())),
                        preferred_element_type=jnp.float32
                        ).astype(jnp.bfloat16)
    t = lax.tanh(h)
    lift_h = h * (t * jnp.bfloat16(0.5) + jnp.bfloat16(0.5))
    eg = lax.dot_general(embT_h, oh, (((1,), (0,)), ((), ())),
                         preferred_element_type=jnp.float32
                         ).astype(jnp.bfloat16)
    h2 = eg + lift_h
    su = h2 * lax.tanh(h2) + h2
    y = lax.dot_general(vT.astype(jnp.bfloat16), su,
                        (((1,), (0,)), ((), ())),
                        preferred_element_type=jnp.float32)
    y32 = jnp.concatenate(
        [y[:, g * LANES:(g + 1) * LANES] for g in range(R)], axis=0)
    grow = pl.program_id(0) * R + lax.broadcasted_iota(jnp.int32, (R, 1), 0)
    y_ref[...] = jnp.where(grow < N_ROWS_REAL, y32, 0.0)


def _atom_scalars(z2, p3, emb, wp, wg, w2):
    return pl.pallas_call(
        _atom_scalar_body,
        grid=(NB,),
        in_specs=[
            pl.BlockSpec((R, LANES), lambda i: (i, 0)),
            pl.BlockSpec((3, R, LANES), lambda i: (0, i, 0)),
            pl.BlockSpec((ZMAX, D), lambda i: (0, 0)),
            pl.BlockSpec((3, D), lambda i: (0, 0)),
            pl.BlockSpec((D, 1), lambda i: (0, 0)),
            pl.BlockSpec((D, 1), lambda i: (0, 0)),
        ],
        out_specs=pl.BlockSpec((R, LANES), lambda i: (i, 0)),
        out_shape=jax.ShapeDtypeStruct((ROWS, LANES), jnp.float32),
    )(z2, p3, emb, wp, wg, w2)


def _segsum_body(y_hbm, idx_hbm, init_hbm, out_hbm, yv, iv, acc):
    s = lax.axis_index("s")

    pltpu.sync_copy(y_hbm.at[s], yv)
    pltpu.sync_copy(idx_hbm.at[s], iv)

    @pl.when(s == 0)
    def _():
        pltpu.sync_copy(init_hbm, acc)

    plsc.subcore_barrier()

    pltpu.sync_copy(yv, acc.at[iv], add=True)

    plsc.subcore_barrier()

    @pl.when(s == 0)
    def _():
        pltpu.sync_copy(acc, out_hbm)


@functools.cache
def _build_segsum():
    return pl.kernel(
        _segsum_body,
        out_type=jax.ShapeDtypeStruct((NUM_SEG,), jnp.float32),
        mesh=plsc.VectorSubcoreMesh(core_axis_name="c", subcore_axis_name="s",
                                    num_cores=1, num_subcores=NUM_TILES),
        scratch_types=[
            pltpu.VMEM((K_WIN * LANES,), jnp.float32),
            pltpu.VMEM((K_WIN * LANES,), jnp.int32),
            pltpu.VMEM_SHARED((NUM_SEG,), jnp.float32),
        ],
    )


def kernel(z, pos, batch, embedding, Wp, w_gate, W2, b2):
    pad = N_PAD - N
    z2 = jnp.pad(z.astype(jnp.int32), (0, pad)).reshape(ROWS, LANES)
    p3 = jnp.pad(pos.T, ((0, 0), (0, pad))).reshape(3, ROWS, LANES)
    wg = w_gate.reshape(D, 1)

    y = _atom_scalars(z2, p3, embedding, Wp, wg, W2)

    y3 = y.reshape(NUM_TILES, K_WIN * LANES)
    idx3 = jnp.pad(batch.astype(jnp.int32), (0, pad)).reshape(
        NUM_TILES, K_WIN * LANES)
    init = jnp.broadcast_to(b2, (NUM_SEG,)).astype(jnp.float32)

    out = _build_segsum()(y3, idx3, init)
    return out.reshape(NUM_SEG, 1)


# speedup vs baseline: 1.1543x; 1.0491x over previous
import functools

import jax
import jax.numpy as jnp
from jax import lax
from jax.experimental import pallas as pl
from jax.experimental.pallas import tpu as pltpu
from jax.experimental.pallas import tpu_sc as plsc

N = 320000
NUM_SEG = 10000
D = 128
ZMAX = 100

NUM_TILES = 16
LANES = 128
ROWS = 2560
N_PAD = ROWS * LANES
R = 256
NB = ROWS // R
K_WIN = ROWS // NUM_TILES
N_ROWS_REAL = N // LANES


B = R * LANES


def _atom_scalar_body(z_ref, p3_ref, emb_ref, wp_ref,
                      wg_ref, w2_ref, y_ref):
    embT_h = (emb_ref[...].T * 0.5).astype(jnp.bfloat16)
    wpT_h = wp_ref[...].T * 0.5
    vT = (wg_ref[...] * w2_ref[...]).T
    types = lax.broadcasted_iota(jnp.int32, (ZMAX, LANES), 0)

    oh = jnp.concatenate(
        [(types == z_ref[g:g + 1, :]) for g in range(R)],
        axis=1).astype(jnp.bfloat16)
    pose = jnp.concatenate(
        [jnp.concatenate([p3_ref[k, g:g + 1, :] for g in range(R)], axis=1)
         for k in range(3)], axis=0)

    h = lax.dot_general(wpT_h, pose, (((1,), (0,)), ((), ())),
                        preferred_element_type=jnp.float32
                        ).astype(jnp.bfloat16)
    t = lax.tanh(h)
    lift_h = h * (t * jnp.bfloat16(0.5) + jnp.bfloat16(0.5))
    eg = lax.dot_general(embT_h, oh, (((1,), (0,)), ((), ())),
                         preferred_element_type=jnp.float32
                         ).astype(jnp.bfloat16)
    h2 = eg + lift_h
    su = h2 * lax.tanh(h2) + h2
    y = lax.dot_general(vT.astype(jnp.bfloat16), su,
                        (((1,), (0,)), ((), ())),
                        preferred_element_type=jnp.float32)
    y32 = jnp.concatenate(
        [y[:, g * LANES:(g + 1) * LANES] for g in range(R)], axis=0)
    grow = pl.program_id(0) * R + lax.broadcasted_iota(jnp.int32, (R, 1), 0)
    y_ref[...] = jnp.where(grow < N_ROWS_REAL, y32, 0.0)


def _atom_scalars(z2, p3, emb, wp, wg, w2):
    return pl.pallas_call(
        _atom_scalar_body,
        grid=(NB,),
        in_specs=[
            pl.BlockSpec((R, LANES), lambda i: (i, 0)),
            pl.BlockSpec((3, R, LANES), lambda i: (0, i, 0)),
            pl.BlockSpec((ZMAX, D), lambda i: (0, 0)),
            pl.BlockSpec((3, D), lambda i: (0, 0)),
            pl.BlockSpec((D, 1), lambda i: (0, 0)),
            pl.BlockSpec((D, 1), lambda i: (0, 0)),
        ],
        out_specs=pl.BlockSpec((R, LANES), lambda i: (i, 0)),
        out_shape=jax.ShapeDtypeStruct((ROWS, LANES), jnp.float32),
    )(z2, p3, emb, wp, wg, w2)


def _segsum_body(y_hbm, idx_hbm, init_hbm, out_hbm, yv, iv, acc):
    c = lax.axis_index("c")
    s = lax.axis_index("s")

    pltpu.sync_copy(y_hbm.at[c, s], yv)
    pltpu.sync_copy(idx_hbm.at[c, s], iv)

    @pl.when(s == 0)
    def _():
        pltpu.sync_copy(init_hbm, acc)

    plsc.subcore_barrier()

    pltpu.sync_copy(yv, acc.at[iv], add=True)

    plsc.subcore_barrier()

    @pl.when(s == 0)
    def _():
        pltpu.sync_copy(acc, out_hbm.at[c])


@functools.cache
def _build_segsum():
    return pl.kernel(
        _segsum_body,
        out_type=jax.ShapeDtypeStruct((2, NUM_SEG), jnp.float32),
        mesh=plsc.VectorSubcoreMesh(core_axis_name="c", subcore_axis_name="s",
                                    num_cores=2, num_subcores=NUM_TILES),
        scratch_types=[
            pltpu.VMEM((N_PAD // 32,), jnp.float32),
            pltpu.VMEM((N_PAD // 32,), jnp.int32),
            pltpu.VMEM_SHARED((NUM_SEG,), jnp.float32),
        ],
    )


def kernel(z, pos, batch, embedding, Wp, w_gate, W2, b2):
    pad = N_PAD - N
    z2 = jnp.pad(z.astype(jnp.int32), (0, pad)).reshape(ROWS, LANES)
    p3 = jnp.pad(pos.T, ((0, 0), (0, pad))).reshape(3, ROWS, LANES)
    wg = w_gate.reshape(D, 1)

    y = _atom_scalars(z2, p3, embedding, Wp, wg, W2)

    y4 = y.reshape(2, NUM_TILES, N_PAD // 32)
    idx4 = jnp.pad(batch.astype(jnp.int32), (0, pad)).reshape(
        2, NUM_TILES, N_PAD // 32)
    init = jnp.broadcast_to(b2 * 0.5, (NUM_SEG,)).astype(jnp.float32)

    out2 = _build_segsum()(y4, idx4, init)
    return (out2[0] + out2[1]).reshape(NUM_SEG, 1)
